# baseline (speedup 1.0000x reference)
import jax
import jax.numpy as jnp
from jax import lax
from jax.experimental import pallas as pl
from jax.experimental.pallas import tpu as pltpu

_N = 20000
_C = 81
_B = (_C - 1) * 4
_RB = 1000
_GRID = _N // _RB


def _body(x_ref, bb_ref, t_ref, bt_ref, out_ref, ce_s, reg_s, cnt_s):
    i = pl.program_id(0)
    x = x_ref[...]
    bb = bb_ref[...]
    bt = bt_ref[0].T
    t = t_ref[0, 0, :].reshape(_RB, 1)

    m = jnp.max(x, axis=1, keepdims=True)
    lse = m + jnp.log(jnp.sum(jnp.exp(x - m), axis=1, keepdims=True))
    cls_iota = lax.broadcasted_iota(jnp.int32, (_RB, _C), 1)
    xt = jnp.sum(jnp.where(cls_iota == t, x, 0.0), axis=1, keepdims=True)
    ce_part = jnp.sum(lse - xt)

    cls = jnp.maximum(t - 1, 0)
    col_iota = lax.broadcasted_iota(jnp.int32, (_RB, _B), 1)
    xm = jnp.where((col_iota >> 2) == cls, bb, 0.0)
    rj = lax.broadcasted_iota(jnp.int32, (_B, 4), 0)
    rk = lax.broadcasted_iota(jnp.int32, (_B, 4), 1)
    sel = jnp.where((rj & 3) == rk, 1.0, 0.0)
    pred = jax.lax.dot_general(
        xm, sel, (((1,), (0,)), ((), ())), preferred_element_type=jnp.float32
    )

    d = pred - bt
    ad = jnp.abs(d)
    sl1 = jnp.where(ad < 1.0, 0.5 * d * d, ad - 0.5)
    fg = (t > 0).astype(jnp.float32)
    reg_part = jnp.sum(sl1 * fg)
    cnt_part = jnp.sum(fg)

    @pl.when(i == 0)
    def _():
        ce_s[0] = 0.0
        reg_s[0] = 0.0
        cnt_s[0] = 0.0

    ce_s[0] += ce_part
    reg_s[0] += reg_part
    cnt_s[0] += cnt_part

    @pl.when(i == _GRID - 1)
    def _():
        ce = ce_s[0] / _N
        fgc = cnt_s[0]
        reg = jnp.where(fgc > 0.0, reg_s[0] / jnp.maximum(fgc, 1.0), 0.0)
        out_ref[0] = ce + reg
        out_ref[1] = ce
        out_ref[2] = reg


_loss = pl.pallas_call(
    _body,
    grid=(_GRID,),
    in_specs=[
        pl.BlockSpec((_RB, _C), lambda i: (i, 0)),
        pl.BlockSpec((_RB, _B), lambda i: (i, 0)),
        pl.BlockSpec((1, 1, _RB), lambda i: (i, 0, 0)),
        pl.BlockSpec((1, 4, _RB), lambda i: (i, 0, 0)),
    ],
    out_specs=pl.BlockSpec(memory_space=pltpu.SMEM),
    out_shape=jax.ShapeDtypeStruct((3,), jnp.float32),
    scratch_shapes=[
        pltpu.SMEM((1,), jnp.float32),
        pltpu.SMEM((1,), jnp.float32),
        pltpu.SMEM((1,), jnp.float32),
    ],
)


def kernel(cls_pred, bbox_pred, cls_targets, bbox_targets):
    bt3 = bbox_targets.reshape(_GRID, _RB, 4).transpose(0, 2, 1)
    out = _loss(
        cls_pred, bbox_pred, cls_targets.reshape(_GRID, 1, _RB), bt3
    )
    return (out[0], out[1], out[2])


# speedup vs baseline: 1.0942x; 1.0942x over previous
import jax
import jax.numpy as jnp
from jax import lax
from jax.experimental import pallas as pl
from jax.experimental.pallas import tpu as pltpu

_N = 20000
_C = 81
_B = (_C - 1) * 4
_RB = 5000
_GRID = _N // _RB


def _body(x_ref, bb_ref, t_ref, bt_ref, out_ref, ce_s, reg_s, cnt_s):
    i = pl.program_id(0)
    x = x_ref[...]
    bb = bb_ref[...]
    bt = bt_ref[0].T
    t = t_ref[0, 0, :].reshape(_RB, 1)

    m = jnp.max(x, axis=1, keepdims=True)
    lse = m + jnp.log(jnp.sum(jnp.exp(x - m), axis=1, keepdims=True))
    cls_iota = lax.broadcasted_iota(jnp.int32, (_RB, _C), 1)
    xt = jnp.sum(jnp.where(cls_iota == t, x, 0.0), axis=1, keepdims=True)
    ce_part = jnp.sum(lse - xt)

    cls = jnp.maximum(t - 1, 0)
    col_iota = lax.broadcasted_iota(jnp.int32, (_RB, _B), 1)
    xm = jnp.where((col_iota >> 2) == cls, bb, 0.0)
    rj = lax.broadcasted_iota(jnp.int32, (_B, 4), 0)
    rk = lax.broadcasted_iota(jnp.int32, (_B, 4), 1)
    sel = jnp.where((rj & 3) == rk, 1.0, 0.0)
    pred = jax.lax.dot_general(
        xm, sel, (((1,), (0,)), ((), ())), preferred_element_type=jnp.float32
    )

    d = pred - bt
    ad = jnp.abs(d)
    sl1 = jnp.where(ad < 1.0, 0.5 * d * d, ad - 0.5)
    fg = (t > 0).astype(jnp.float32)
    reg_part = jnp.sum(sl1 * fg)
    cnt_part = jnp.sum(fg)

    @pl.when(i == 0)
    def _():
        ce_s[0] = 0.0
        reg_s[0] = 0.0
        cnt_s[0] = 0.0

    ce_s[0] += ce_part
    reg_s[0] += reg_part
    cnt_s[0] += cnt_part

    @pl.when(i == _GRID - 1)
    def _():
        ce = ce_s[0] / _N
        fgc = cnt_s[0]
        reg = jnp.where(fgc > 0.0, reg_s[0] / jnp.maximum(fgc, 1.0), 0.0)
        out_ref[0] = ce + reg
        out_ref[1] = ce
        out_ref[2] = reg


_loss = pl.pallas_call(
    _body,
    grid=(_GRID,),
    in_specs=[
        pl.BlockSpec((_RB, _C), lambda i: (i, 0)),
        pl.BlockSpec((_RB, _B), lambda i: (i, 0)),
        pl.BlockSpec((1, 1, _RB), lambda i: (i, 0, 0)),
        pl.BlockSpec((1, 4, _RB), lambda i: (i, 0, 0)),
    ],
    out_specs=pl.BlockSpec(memory_space=pltpu.SMEM),
    out_shape=jax.ShapeDtypeStruct((3,), jnp.float32),
    scratch_shapes=[
        pltpu.SMEM((1,), jnp.float32),
        pltpu.SMEM((1,), jnp.float32),
        pltpu.SMEM((1,), jnp.float32),
    ],
)


def kernel(cls_pred, bbox_pred, cls_targets, bbox_targets):
    bt3 = bbox_targets.reshape(_GRID, _RB, 4).transpose(0, 2, 1)
    out = _loss(
        cls_pred, bbox_pred, cls_targets.reshape(_GRID, 1, _RB), bt3
    )
    return (out[0], out[1], out[2])


# speedup vs baseline: 1.0944x; 1.0002x over previous
import jax
import jax.numpy as jnp
from jax import lax
from jax.experimental import pallas as pl
from jax.experimental.pallas import tpu as pltpu

_N = 20000
_C = 81
_B = (_C - 1) * 4
_RB = 4000
_HB = _RB // 2
_GRID = _N // _RB


def _body(x_ref, bb0_ref, bb1_ref, t_ref, bt_ref, out_ref, ce_s, reg_s, cnt_s):
    i = pl.program_id(0)
    x = x_ref[...]
    bt = bt_ref[0].T
    t = t_ref[0, 0, :].reshape(_RB, 1)

    m = jnp.max(x, axis=1, keepdims=True)
    lse = m + jnp.log(jnp.sum(jnp.exp(x - m), axis=1, keepdims=True))
    cls_iota = lax.broadcasted_iota(jnp.int32, (_RB, _C), 1)
    xt = jnp.sum(jnp.where(cls_iota == t, x, 0.0), axis=1, keepdims=True)
    ce_part = jnp.sum(lse - xt)

    cls = jnp.maximum(t - 1, 0)
    col_iota = lax.broadcasted_iota(jnp.int32, (_HB, _B), 1)
    rj = lax.broadcasted_iota(jnp.int32, (_B, 4), 0)
    rk = lax.broadcasted_iota(jnp.int32, (_B, 4), 1)
    sel = jnp.where((rj & 3) == rk, 1.0, 0.0)
    preds = []
    for h, bb_ref in enumerate((bb0_ref, bb1_ref)):
        bb = bb_ref[...]
        clsh = lax.slice_in_dim(cls, h * _HB, (h + 1) * _HB, axis=0)
        xm = jnp.where((col_iota >> 2) == clsh, bb, 0.0)
        preds.append(jax.lax.dot_general(
            xm, sel, (((1,), (0,)), ((), ())),
            preferred_element_type=jnp.float32,
        ))
    pred = jnp.concatenate(preds, axis=0)

    d = pred - bt
    ad = jnp.abs(d)
    sl1 = jnp.where(ad < 1.0, 0.5 * d * d, ad - 0.5)
    fg = (t > 0).astype(jnp.float32)
    reg_part = jnp.sum(sl1 * fg)
    cnt_part = jnp.sum(fg)

    @pl.when(i == 0)
    def _():
        ce_s[0] = 0.0
        reg_s[0] = 0.0
        cnt_s[0] = 0.0

    ce_s[0] += ce_part
    reg_s[0] += reg_part
    cnt_s[0] += cnt_part

    @pl.when(i == _GRID - 1)
    def _():
        ce = ce_s[0] / _N
        fgc = cnt_s[0]
        reg = jnp.where(fgc > 0.0, reg_s[0] / jnp.maximum(fgc, 1.0), 0.0)
        out_ref[0] = ce + reg
        out_ref[1] = ce
        out_ref[2] = reg


_loss = pl.pallas_call(
    _body,
    grid=(_GRID,),
    in_specs=[
        pl.BlockSpec((_RB, _C), lambda i: (i, 0)),
        pl.BlockSpec((_HB, _B), lambda i: (2 * i, 0)),
        pl.BlockSpec((_HB, _B), lambda i: (2 * i + 1, 0)),
        pl.BlockSpec((1, 1, _RB), lambda i: (i, 0, 0)),
        pl.BlockSpec((1, 4, _RB), lambda i: (i, 0, 0)),
    ],
    out_specs=pl.BlockSpec(memory_space=pltpu.SMEM),
    out_shape=jax.ShapeDtypeStruct((3,), jnp.float32),
    scratch_shapes=[
        pltpu.SMEM((1,), jnp.float32),
        pltpu.SMEM((1,), jnp.float32),
        pltpu.SMEM((1,), jnp.float32),
    ],
)


def kernel(cls_pred, bbox_pred, cls_targets, bbox_targets):
    bt3 = bbox_targets.reshape(_GRID, _RB, 4).transpose(0, 2, 1)
    out = _loss(
        cls_pred, bbox_pred, bbox_pred,
        cls_targets.reshape(_GRID, 1, _RB), bt3
    )
    return (out[0], out[1], out[2])


# speedup vs baseline: 1.0976x; 1.0029x over previous
import jax
import jax.numpy as jnp
from jax import lax
from jax.experimental import pallas as pl
from jax.experimental.pallas import tpu as pltpu

_N = 20000
_C = 81
_B = (_C - 1) * 4
_RB = 4000
_GRID = _N // _RB


def _body(x_ref, bb_ref, t_ref, bt_ref, out_ref, ce_s, reg_s, cnt_s):
    i = pl.program_id(0)
    x = x_ref[...]
    bb = bb_ref[...]
    bt = bt_ref[0].T
    t = t_ref[0, 0, :].reshape(_RB, 1)

    m = jnp.max(x, axis=1, keepdims=True)
    lse = m + jnp.log(jnp.sum(jnp.exp(x - m), axis=1, keepdims=True))
    cls_iota = lax.broadcasted_iota(jnp.int32, (_RB, _C), 1)
    xt = jnp.sum(jnp.where(cls_iota == t, x, 0.0), axis=1, keepdims=True)
    ce_part = jnp.sum(lse - xt)

    cls = jnp.maximum(t - 1, 0)
    col_iota = lax.broadcasted_iota(jnp.int32, (_RB, _B), 1)
    xm = jnp.where((col_iota >> 2) == cls, bb, 0.0)
    rj = lax.broadcasted_iota(jnp.int32, (_B, 4), 0)
    rk = lax.broadcasted_iota(jnp.int32, (_B, 4), 1)
    sel = jnp.where((rj & 3) == rk, 1.0, 0.0)
    pred = jax.lax.dot_general(
        xm, sel, (((1,), (0,)), ((), ())), preferred_element_type=jnp.float32
    )

    d = pred - bt
    ad = jnp.abs(d)
    sl1 = jnp.where(ad < 1.0, 0.5 * d * d, ad - 0.5)
    fg = (t > 0).astype(jnp.float32)
    reg_part = jnp.sum(sl1 * fg)
    cnt_part = jnp.sum(fg)

    @pl.when(i == 0)
    def _():
        ce_s[0] = 0.0
        reg_s[0] = 0.0
        cnt_s[0] = 0.0

    ce_s[0] += ce_part
    reg_s[0] += reg_part
    cnt_s[0] += cnt_part

    @pl.when(i == _GRID - 1)
    def _():
        ce = ce_s[0] / _N
        fgc = cnt_s[0]
        reg = jnp.where(fgc > 0.0, reg_s[0] / jnp.maximum(fgc, 1.0), 0.0)
        out_ref[0] = ce + reg
        out_ref[1] = ce
        out_ref[2] = reg


_loss = pl.pallas_call(
    _body,
    grid=(_GRID,),
    in_specs=[
        pl.BlockSpec((_RB, _C), lambda i: (i, 0)),
        pl.BlockSpec((_RB, _B), lambda i: (i, 0)),
        pl.BlockSpec((1, 1, _RB), lambda i: (i, 0, 0)),
        pl.BlockSpec((1, 4, _RB), lambda i: (i, 0, 0)),
    ],
    out_specs=pl.BlockSpec(memory_space=pltpu.SMEM),
    out_shape=jax.ShapeDtypeStruct((3,), jnp.float32),
    scratch_shapes=[
        pltpu.SMEM((1,), jnp.float32),
        pltpu.SMEM((1,), jnp.float32),
        pltpu.SMEM((1,), jnp.float32),
    ],
)


def kernel(cls_pred, bbox_pred, cls_targets, bbox_targets):
    bt3 = bbox_targets.reshape(_GRID, _RB, 4).transpose(0, 2, 1)
    out = _loss(
        cls_pred, bbox_pred, cls_targets.reshape(_GRID, 1, _RB), bt3
    )
    return (out[0], out[1], out[2])
